# baseline (speedup 1.0000x reference)
import functools

import jax
import jax.numpy as jnp
from jax import lax
from jax.experimental import pallas as pl
from jax.experimental.pallas import tpu as pltpu
from jax.experimental.pallas import tpu_sc as plsc

_DIM = 64
_R = 8
_SCALING = 2.0
_L = 16
_NB = 32768
_CW = 256
_W = 18432

_DNUMS = lax.GatherDimensionNumbers(
    offset_dims=(), collapsed_slice_dims=(0,), start_index_map=(0,))


def _perm(x, idx):
    return lax.gather(x, idx[:, None], _DNUMS, slice_sizes=(1,),
                      mode=lax.GatherScatterMode.PROMISE_IN_BOUNDS)


def _lane_total(x, iota):
    for k in (8, 4, 2, 1):
        x = x + _perm(x, iota ^ k)
    return x


def _tc_body(wo_ref, lbt_ref, wt_ref, at_ref, u_ref):
    vb = jnp.dot(wo_ref[...], lbt_ref[...],
                 preferred_element_type=jnp.float32) * _SCALING
    u = jnp.dot(wo_ref[...], wt_ref[...], preferred_element_type=jnp.float32)
    u = u + jnp.dot(vb, at_ref[...], preferred_element_type=jnp.float32)
    u_ref[...] = u


def _sc_mv_body(nc, wt_hbm, at_hbm, lb_hbm, wo_hbm, usc_hbm,
                wt_b0, wt_b1, at_b0, at_b1, lb_v, wo_v, wbc_v, u_v,
                sem0, sem1):
    wid = lax.axis_index("s") * nc + lax.axis_index("c")
    base = wid * _W
    n_ch = _W // _CW
    n_g = _CW // _L

    def start(c, wt_buf, at_buf, sem):
        b = base + c * _CW
        pltpu.async_copy(wt_hbm.at[pl.ds(0, _DIM), pl.ds(b, _CW)], wt_buf, sem)
        pltpu.async_copy(at_hbm.at[pl.ds(0, _R), pl.ds(b, _CW)], at_buf, sem)

    def drain(wt_buf, at_buf, sem):
        pltpu.make_async_copy(
            wt_hbm.at[pl.ds(0, _DIM), pl.ds(0, _CW)], wt_buf, sem).wait()
        pltpu.make_async_copy(
            at_hbm.at[pl.ds(0, _R), pl.ds(0, _CW)], at_buf, sem).wait()

    start(0, wt_b0, at_b0, sem0)
    pltpu.sync_copy(wo_hbm, wo_v)
    pltpu.sync_copy(lb_hbm, lb_v)

    iota = lax.iota(jnp.int32, _L)
    wch = [wo_v[pl.ds(16 * c, 16)] for c in range(_DIM // _L)]
    for d in range(_DIM):
        wbc_v[pl.ds(d * _L, _L)] = _perm(
            wch[d // _L], jnp.full((_L,), d % _L, jnp.int32))
    for j in range(_R):
        acc = lb_v[j, pl.ds(0, 16)] * wch[0]
        for c in range(1, _DIM // _L):
            acc = acc + lb_v[j, pl.ds(16 * c, 16)] * wch[c]
        wbc_v[pl.ds((_DIM + j) * _L, _L)] = _lane_total(acc, iota) * _SCALING

    def compute(wt_buf, at_buf, c):
        accs = [jnp.zeros((_L,), jnp.float32)] * n_g
        for d in range(_DIM):
            wb = wbc_v[pl.ds(d * _L, _L)]
            for g in range(n_g):
                accs[g] = accs[g] + wt_buf[d, pl.ds(16 * g, 16)] * wb
        for j in range(_R):
            vb = wbc_v[pl.ds((_DIM + j) * _L, _L)]
            for g in range(n_g):
                accs[g] = accs[g] + at_buf[j, pl.ds(16 * g, 16)] * vb
        for g in range(n_g):
            u_v[pl.ds(c * _CW + 16 * g, 16)] = accs[g]

    def pair(t, carry):
        c0 = 2 * t
        start(c0 + 1, wt_b1, at_b1, sem1)
        drain(wt_b0, at_b0, sem0)
        compute(wt_b0, at_b0, c0)

        @pl.when(c0 + 2 < n_ch)
        def _():
            start(c0 + 2, wt_b0, at_b0, sem0)

        drain(wt_b1, at_b1, sem1)
        compute(wt_b1, at_b1, c0 + 1)
        return carry

    lax.fori_loop(0, n_ch // 2, pair, 0)
    pltpu.sync_copy(u_v, usc_hbm.at[pl.ds(base, _W)])


def _sc_body(nc, b_per_w, idx_hbm, u_hbm, bias_hbm, out_hbm,
             idx_v, g_v, bias_v, sem):
    wid = lax.axis_index("s") * nc + lax.axis_index("c")
    n_chunks = b_per_w // 128

    pltpu.sync_copy(idx_hbm.at[wid], idx_v)
    copies = []
    for j in range(n_chunks):
        copies.append(pltpu.async_copy(
            u_hbm.at[idx_v.at[j]], g_v.at[pl.ds(j * 128, 128)], sem))
    pltpu.sync_copy(bias_hbm, bias_v)
    bias = bias_v[...]
    for cp in copies:
        cp.wait()

    def chunk(t, carry):
        z = g_v[pl.ds(t * _L, _L)] + bias
        g_v[pl.ds(t * _L, _L)] = 1.0 / (1.0 + jnp.exp(-z))
        return carry

    lax.fori_loop(0, b_per_w // _L, chunk, 0)
    pltpu.sync_copy(g_v, out_hbm.at[pl.ds(wid * b_per_w, b_per_w)])


def kernel(item_indices, weight, lora_A, lora_B, W_out, b_out):
    batch = item_indices.shape[0]
    num_items = weight.shape[0]
    info = plsc.get_sparse_core_info()
    nc, ns = info.num_cores, info.num_subcores
    nw = nc * ns
    b_per_w = batch // nw
    assert batch % (nw * 128) == 0

    idx3 = item_indices.astype(jnp.int32).reshape(nw, b_per_w // 128, 128)
    wt = weight.T
    at = lora_A.T
    lbt = lora_B.T
    wo = W_out
    bias = jnp.broadcast_to(b_out, (_L,))

    sc_items = nw * _W
    tc_items = num_items - sc_items
    assert sc_items % _NB == 0 and tc_items > 0
    koff = sc_items // _NB
    n_blocks = (tc_items + _NB - 1) // _NB

    mesh = plsc.VectorSubcoreMesh(core_axis_name="c", subcore_axis_name="s")

    u_sc = functools.partial(
        pl.kernel,
        mesh=mesh,
        out_type=jax.ShapeDtypeStruct((sc_items,), jnp.float32),
        scratch_types=[
            pltpu.VMEM((_DIM, _CW), jnp.float32),
            pltpu.VMEM((_DIM, _CW), jnp.float32),
            pltpu.VMEM((_R, _CW), jnp.float32),
            pltpu.VMEM((_R, _CW), jnp.float32),
            pltpu.VMEM((_R, _DIM), jnp.float32),
            pltpu.VMEM((_DIM,), jnp.float32),
            pltpu.VMEM(((_DIM + _R) * _L,), jnp.float32),
            pltpu.VMEM((_W,), jnp.float32),
            pltpu.SemaphoreType.DMA,
            pltpu.SemaphoreType.DMA,
        ],
    )(functools.partial(_sc_mv_body, nc))(wt, at, lora_B, wo.reshape(_DIM))

    u_tc = pl.pallas_call(
        _tc_body,
        grid=(n_blocks,),
        in_specs=[
            pl.BlockSpec((1, _DIM), lambda c: (0, 0)),
            pl.BlockSpec((_DIM, _R), lambda c: (0, 0)),
            pl.BlockSpec((_DIM, _NB), lambda c: (0, c + koff)),
            pl.BlockSpec((_R, _NB), lambda c: (0, c + koff)),
        ],
        out_specs=pl.BlockSpec((1, _NB), lambda c: (0, c)),
        out_shape=jax.ShapeDtypeStruct((1, tc_items), jnp.float32),
    )(wo, lbt, wt, at)

    u = jnp.concatenate([u_sc, u_tc.reshape(tc_items)])

    sc_call = functools.partial(
        pl.kernel,
        mesh=mesh,
        out_type=jax.ShapeDtypeStruct((batch,), jnp.float32),
        scratch_types=[
            pltpu.VMEM((b_per_w // 128, 128), jnp.int32),
            pltpu.VMEM((b_per_w,), jnp.float32),
            pltpu.VMEM((_L,), jnp.float32),
            pltpu.SemaphoreType.DMA,
        ],
    )(functools.partial(_sc_body, nc, b_per_w))
    out = sc_call(idx3, u, bias)
    return out.reshape(batch, 1)


# speedup vs baseline: 2.0759x; 2.0759x over previous
import functools

import jax
import jax.numpy as jnp
from jax import lax
from jax.experimental import pallas as pl
from jax.experimental.pallas import tpu as pltpu
from jax.experimental.pallas import tpu_sc as plsc

_DIM = 64
_R = 8
_SCALING = 2.0
_L = 16
_NB = 32768


def _tc_body(wo_ref, lbt_ref, wt_ref, at_ref, u_ref):
    vb = jnp.dot(wo_ref[...], lbt_ref[...],
                 preferred_element_type=jnp.float32) * _SCALING
    u = jnp.dot(wo_ref[...], wt_ref[...], preferred_element_type=jnp.float32)
    u = u + jnp.dot(vb, at_ref[...], preferred_element_type=jnp.float32)
    u_ref[...] = u


def _sc_body(nc, b_per_w, idx_hbm, u_hbm, bias_hbm, out_hbm,
             idx_v, g_v, bias_v, sem):
    wid = lax.axis_index("s") * nc + lax.axis_index("c")
    n_chunks = b_per_w // 128

    pltpu.sync_copy(idx_hbm.at[wid], idx_v)
    copies = []
    for j in range(n_chunks):
        copies.append(pltpu.async_copy(
            u_hbm.at[idx_v.at[j]], g_v.at[pl.ds(j * 128, 128)], sem))
    pltpu.sync_copy(bias_hbm, bias_v)
    bias = bias_v[...]
    for cp in copies:
        cp.wait()

    def chunk(t, carry):
        z = g_v[pl.ds(t * _L, _L)] + bias
        g_v[pl.ds(t * _L, _L)] = 1.0 / (1.0 + jnp.exp(-z))
        return carry

    lax.fori_loop(0, b_per_w // _L, chunk, 0)
    pltpu.sync_copy(g_v, out_hbm.at[pl.ds(wid * b_per_w, b_per_w)])


def kernel(item_indices, weight, lora_A, lora_B, W_out, b_out):
    batch = item_indices.shape[0]
    num_items = weight.shape[0]
    info = plsc.get_sparse_core_info()
    nc, ns = info.num_cores, info.num_subcores
    nw = nc * ns
    b_per_w = batch // nw
    assert batch % (nw * 128) == 0

    idx3 = item_indices.astype(jnp.int32).reshape(nw, b_per_w // 128, 128)
    wt = weight.T
    at = lora_A.T
    lbt = lora_B.T
    wo = W_out
    bias = jnp.broadcast_to(b_out, (_L,))

    n_blocks = (num_items + _NB - 1) // _NB
    u = pl.pallas_call(
        _tc_body,
        grid=(n_blocks,),
        in_specs=[
            pl.BlockSpec((1, _DIM), lambda c: (0, 0)),
            pl.BlockSpec((_DIM, _R), lambda c: (0, 0)),
            pl.BlockSpec((_DIM, _NB), lambda c: (0, c)),
            pl.BlockSpec((_R, _NB), lambda c: (0, c)),
        ],
        out_specs=pl.BlockSpec((1, _NB), lambda c: (0, c)),
        out_shape=jax.ShapeDtypeStruct((1, num_items), jnp.float32),
    )(wo, lbt, wt, at)
    u = u.reshape(num_items)

    mesh = plsc.VectorSubcoreMesh(core_axis_name="c", subcore_axis_name="s")
    sc_call = functools.partial(
        pl.kernel,
        mesh=mesh,
        out_type=jax.ShapeDtypeStruct((batch,), jnp.float32),
        scratch_types=[
            pltpu.VMEM((b_per_w // 128, 128), jnp.int32),
            pltpu.VMEM((b_per_w,), jnp.float32),
            pltpu.VMEM((_L,), jnp.float32),
            pltpu.SemaphoreType.DMA,
        ],
    )(functools.partial(_sc_body, nc, b_per_w))
    out = sc_call(idx3, u, bias)
    return out.reshape(batch, 1)
